# Initial kernel scaffold; baseline (speedup 1.0000x reference)
import jax
import jax.numpy as jnp
from jax.experimental import pallas as pl


def kernel(x, edge_index, edge_weight, W, b):
    raise NotImplementedError("write your pallas kernel here")


# speedup vs baseline: 6.0621x; 6.0621x over previous
import functools

import jax
import jax.numpy as jnp
from jax import lax
from jax.experimental import pallas as pl
from jax.experimental.pallas import tpu as pltpu
from jax.experimental.pallas import tpu_sc as plsc

N = 10000
E = 320000
D = 128
L = 16
NC = 2
NS = 16
NW = NC * NS
EPT = E // NW
CHUNK = 80
NCHUNK = EPT // CHUNK
NPAD = 10240
RPT = NPAD // NS


def _sc_body(x_hbm, src_hbm, dst_hbm, ew_hbm, z_hbm, out_hbm,
             src_v, dst_v, ew_v, rows_v, acc, sem):
    cid = lax.axis_index("c")
    sid = lax.axis_index("s")
    wid = cid * NS + sid
    base_e = wid * EPT

    pltpu.sync_copy(z_hbm.at[pl.ds(sid * RPT, RPT)],
                    acc.at[pl.ds(sid * RPT, RPT)])

    pltpu.sync_copy(src_hbm.at[pl.ds(base_e, EPT)], src_v)
    pltpu.sync_copy(ew_hbm.at[pl.ds(base_e, EPT)], ew_v)
    pltpu.sync_copy(dst_hbm.at[wid], dst_v)

    plsc.subcore_barrier()

    def step(i, carry):
        idx = src_v.at[pl.ds(i * CHUNK, CHUNK)]
        pltpu.async_copy(x_hbm.at[idx], rows_v, sem).wait()

        def scale_group(g, c):
            wv = ew_v[pl.ds(i * CHUNK + g * L, L)]
            base_r = g * L
            for j in range(L):
                wj = jnp.full((L,), wv[j], jnp.float32)
                for k in range(D // L):
                    rows_v[base_r + j, pl.ds(k * L, L)] = (
                        rows_v[base_r + j, pl.ds(k * L, L)] * wj)
            return c
        lax.fori_loop(0, CHUNK // L, scale_group, 0)

        pltpu.sync_copy(rows_v, acc.at[dst_v.at[i]], add=True)
        return carry

    lax.fori_loop(0, NCHUNK, step, 0)

    plsc.subcore_barrier()

    pltpu.sync_copy(acc.at[pl.ds(sid * RPT, RPT)],
                    out_hbm.at[cid, pl.ds(sid * RPT, RPT)])


def _sc_aggregate(x, src, dst3, ew, zeros):
    mesh = plsc.VectorSubcoreMesh(core_axis_name="c", subcore_axis_name="s",
                                  num_cores=NC, num_subcores=NS)
    fn = pl.kernel(
        _sc_body,
        out_type=jax.ShapeDtypeStruct((NC, NPAD, D), jnp.float32),
        mesh=mesh,
        scratch_types=[
            pltpu.VMEM((EPT,), jnp.int32),
            pltpu.VMEM((NCHUNK, CHUNK), jnp.int32),
            pltpu.VMEM((EPT,), jnp.float32),
            pltpu.VMEM((CHUNK, D), jnp.float32),
            pltpu.VMEM_SHARED((NPAD, D), jnp.float32),
            pltpu.SemaphoreType.DMA,
        ],
    )
    return fn(x, src, dst3, ew, zeros)


def _tc_body(p0_ref, p1_ref, w_ref, b_ref, o_ref):
    s = p0_ref[...] + p1_ref[...]
    o_ref[...] = (jnp.dot(s, w_ref[...], preferred_element_type=jnp.float32)
                  + b_ref[...])


def _tc_combine(p0, p1, W, b2):
    BM = 1000
    grid = (N // BM,)
    return pl.pallas_call(
        _tc_body,
        grid=grid,
        in_specs=[
            pl.BlockSpec((BM, D), lambda i: (i, 0)),
            pl.BlockSpec((BM, D), lambda i: (i, 0)),
            pl.BlockSpec((D, D), lambda i: (0, 0)),
            pl.BlockSpec((1, D), lambda i: (0, 0)),
        ],
        out_specs=pl.BlockSpec((BM, D), lambda i: (i, 0)),
        out_shape=jax.ShapeDtypeStruct((N, D), jnp.float32),
    )(p0, p1, W, b2)


def kernel(x, edge_index, edge_weight, W, b):
    src = edge_index[0].astype(jnp.int32)
    dst3 = edge_index[1].astype(jnp.int32).reshape(NW, NCHUNK, CHUNK)
    zeros = jnp.zeros((NPAD, D), jnp.float32)
    partial = _sc_aggregate(x, src, dst3, edge_weight, zeros)
    return _tc_combine(partial[0], partial[1], W, b.reshape(1, D))


# speedup vs baseline: 9.0421x; 1.4916x over previous
import functools

import jax
import jax.numpy as jnp
from jax import lax
from jax.experimental import pallas as pl
from jax.experimental.pallas import tpu as pltpu
from jax.experimental.pallas import tpu_sc as plsc

N = 10000
E = 320000
D = 128
L = 16
NC = 2
NS = 16
NW = NC * NS
EPT = E // NW
CHUNK = 80
NCHUNK = EPT // CHUNK
NPAD = 10240
RPT = NPAD // NS
NSUP = 5
SUPC = NCHUNK // NSUP
SUPE = EPT // NSUP


def _sc_body(x_hbm, src_hbm, dst_hbm, ew_hbm, z_hbm, out_hbm,
             src_v, dst_v, ew_v, g0, g1, acc, gsem0, gsem1):
    cid = lax.axis_index("c")
    sid = lax.axis_index("s")
    wid = cid * NS + sid
    base_e = wid * EPT

    pltpu.sync_copy(z_hbm.at[pl.ds(sid * RPT, RPT)],
                    acc.at[pl.ds(sid * RPT, RPT)])

    def load_meta(sup):
        pltpu.sync_copy(src_hbm.at[pl.ds(base_e + sup * SUPE, SUPE)], src_v)
        pltpu.sync_copy(ew_hbm.at[pl.ds(base_e + sup * SUPE, SUPE)], ew_v)
        pltpu.sync_copy(dst_hbm.at[wid, sup], dst_v)

    load_meta(0)
    plsc.subcore_barrier()

    def g_desc(io, gbuf, gsem):
        idx = src_v.at[pl.ds(io * CHUNK, CHUNK)]
        return pltpu.make_async_copy(x_hbm.at[idx], gbuf, gsem)

    def scale(io, buf):
        def scale_group(g, c):
            wv = ew_v[pl.ds(io * CHUNK + g * L, L)]
            base_r = g * L
            for j in range(L):
                wj = jnp.full((L,), wv[j], jnp.float32)
                for k in range(D // L):
                    buf[base_r + j, pl.ds(k * L, L)] = (
                        buf[base_r + j, pl.ds(k * L, L)] * wj)
            return c
        lax.fori_loop(0, CHUNK // L, scale_group, 0)

    def stage(i, bufA, semA, bufB, semB):
        io = lax.rem(i, SUPC)

        @pl.when(io != SUPC - 1)
        def _():
            g_desc(io + 1, bufB, semB).start()

        g_desc(io, bufA, semA).wait()
        scale(io, bufA)
        pltpu.sync_copy(bufA, acc.at[dst_v.at[io]], add=True)

        @pl.when(jnp.logical_and(io == SUPC - 1, i != NCHUNK - 1))
        def _():
            load_meta((i + 1) // SUPC)
            g_desc(0, bufB, semB).start()

    g_desc(0, g0, gsem0).start()

    def pair(t, c):
        stage(2 * t, g0, gsem0, g1, gsem1)
        stage(2 * t + 1, g1, gsem1, g0, gsem0)
        return c
    lax.fori_loop(0, (NCHUNK - 1) // 2, pair, 0)
    stage(NCHUNK - 1, g0, gsem0, g1, gsem1)

    plsc.subcore_barrier()

    pltpu.sync_copy(acc.at[pl.ds(sid * RPT, RPT)],
                    out_hbm.at[cid, pl.ds(sid * RPT, RPT)])


def _sc_aggregate(x, src, dst3, ew, zeros):
    mesh = plsc.VectorSubcoreMesh(core_axis_name="c", subcore_axis_name="s",
                                  num_cores=NC, num_subcores=NS)
    fn = pl.kernel(
        _sc_body,
        out_type=jax.ShapeDtypeStruct((NC, NPAD, D), jnp.float32),
        mesh=mesh,
        scratch_types=[
            pltpu.VMEM((SUPE,), jnp.int32),
            pltpu.VMEM((SUPC, CHUNK), jnp.int32),
            pltpu.VMEM((SUPE,), jnp.float32),
            pltpu.VMEM((CHUNK, D), jnp.float32),
            pltpu.VMEM((CHUNK, D), jnp.float32),
            pltpu.VMEM_SHARED((NPAD, D), jnp.float32),
            pltpu.SemaphoreType.DMA,
            pltpu.SemaphoreType.DMA,
        ],
    )
    return fn(x, src, dst3, ew, zeros)


def _tc_body(p0_ref, p1_ref, w_ref, b_ref, o_ref):
    s = p0_ref[...] + p1_ref[...]
    o_ref[...] = (jnp.dot(s, w_ref[...], preferred_element_type=jnp.float32)
                  + b_ref[...])


def _tc_combine(p0, p1, W, b2):
    BM = 1000
    grid = (N // BM,)
    return pl.pallas_call(
        _tc_body,
        grid=grid,
        in_specs=[
            pl.BlockSpec((BM, D), lambda i: (i, 0)),
            pl.BlockSpec((BM, D), lambda i: (i, 0)),
            pl.BlockSpec((D, D), lambda i: (0, 0)),
            pl.BlockSpec((1, D), lambda i: (0, 0)),
        ],
        out_specs=pl.BlockSpec((BM, D), lambda i: (i, 0)),
        out_shape=jax.ShapeDtypeStruct((N, D), jnp.float32),
    )(p0, p1, W, b2)


def kernel(x, edge_index, edge_weight, W, b):
    src = edge_index[0].astype(jnp.int32)
    dst3 = edge_index[1].astype(jnp.int32).reshape(NW, NSUP, SUPC, CHUNK)
    zeros = jnp.zeros((NPAD, D), jnp.float32)
    partial = _sc_aggregate(x, src, dst3, edge_weight, zeros)
    return _tc_combine(partial[0], partial[1], W, b.reshape(1, D))
